# baseline (speedup 1.0000x reference)
import functools

import jax
import jax.numpy as jnp
from jax import lax
from jax.experimental import pallas as pl
from jax.experimental.pallas import tpu as pltpu
from jax.experimental.pallas import tpu_sc as plsc

_INTERPRET = False

_D = 32
_IMG = 64
_C = 3
_NL = 85
_SLOTS = 88
_SORT_W = 96
_CH = 4
_PREC = jax.lax.Precision.DEFAULT


def _decode_body(zw_ref, w_ref, b_ref, out_ref):
    x = jnp.dot(zw_ref[...], w_ref[...],
                preferred_element_type=jnp.float32, precision=_PREC)
    out_ref[...] = jax.nn.sigmoid(x + b_ref[...][None, :])


def _render_body(glyphs_ref, sorted_ref, params_ref,
                 out_ref, canvas_ref, step_ref, done_ref,
                 sxall_ref, syar_ref, sybr_ref):
    b = pl.program_id(0)
    canvas_ref[...] = jnp.zeros(canvas_ref.shape, jnp.float32)
    step_ref[...] = jnp.full(step_ref.shape, jnp.inf, jnp.float32)
    done_ref[0] = 0
    cnt = sorted_ref[b, _SLOTS]

    lane = jax.lax.broadcasted_iota(jnp.int32, (1, 2 * _IMG), 1)
    gxpair = (lane % _IMG).astype(jnp.float32) * (2.0 / (_IMG - 1)) - 1.0
    selx = lane < _IMG
    gyrow = jax.lax.broadcasted_iota(jnp.int32, (1, _IMG), 1).astype(jnp.float32) * (2.0 / (_IMG - 1)) - 1.0
    xp = jax.lax.broadcasted_iota(jnp.int32, (_D, 1), 0).astype(jnp.float32)
    kcol = jax.lax.broadcasted_iota(jnp.int32, (_IMG, 1), 0)
    ksel = kcol < _D
    kmodcol = (kcol % _D).astype(jnp.float32)
    lmaskA = selx.astype(jnp.float32) * jnp.ones((_D, 1), jnp.float32)
    lmaskB = 1.0 - lmaskA
    leftmask = jnp.broadcast_to(selx, (_IMG, 2 * _IMG))
    half = (_D - 1) / 2.0

    pr = params_ref[0]
    cxa, cya, rwa, rha, offa = (pr[:, i:i + 1] for i in range(5))
    cxb, cyb, rwb, rhb, offb = (pr[:, i:i + 1] for i in range(5, 10))
    sx_a = ((gxpair - (2.0 * cxa - 1.0)) * rwa + 1.0) * half + offa
    sx_b = ((gxpair - (2.0 * cxb - 1.0)) * rwb + 1.0) * half + offb
    sxall_ref[...] = jnp.where(selx, sx_a, sx_b)
    syar_ref[...] = ((gyrow - (2.0 * cya - 1.0)) * rha + 1.0) * half + offa
    sybr_ref[...] = ((gyrow - (2.0 * cyb - 1.0)) * rhb + 1.0) * half + offb

    def chunk(ci, carry):
        @pl.when((ci * _CH < cnt) & (done_ref[0] == 0))
        def _():
            st = step_ref[...]
            cv = [canvas_ref[c] for c in range(_C)]
            new_st = st
            for u in range(_CH):
                k = ci * _CH + u
                loc = sorted_ref[b, k]
                sx_row = sxall_ref[pl.ds(k, 1), :]
                rxt2 = jnp.maximum(0.0, 1.0 - jnp.abs(sx_row - xp))
                sya = syar_ref[pl.ds(k, 1), :]
                syb = sybr_ref[pl.ds(k, 1), :]
                sy_sel = jnp.where(ksel, sya, syb)
                rycat_t = jnp.maximum(0.0, 1.0 - jnp.abs(sy_sel - kmodcol))

                g = glyphs_ref[0, loc]
                a2 = jnp.dot(g, rxt2, preferred_element_type=jnp.float32,
                             precision=_PREC)
                r2 = []
                for c in range(_C):
                    a2c = a2[c * _D:(c + 1) * _D, :]
                    aext = jnp.concatenate([a2c * lmaskA, a2c * lmaskB], axis=0)
                    r2.append(jax.lax.dot_general(
                        rycat_t, aext, (((0,), (0,)), ((), ())),
                        preferred_element_type=jnp.float32,
                        precision=_PREC))
                upd = (r2[0] != 0.0) & (new_st == jnp.inf)
                for c in range(_C):
                    cv[c] = jnp.where(upd, r2[c], cv[c])
                new_st = jnp.where(upd, jnp.float32(k), new_st)
            for c in range(_C):
                canvas_ref[c] = cv[c]
            step_ref[...] = new_st
            maxleft = jnp.max(jnp.where(leftmask, new_st, -jnp.inf))
            done_ref[0] = jnp.where(maxleft < jnp.inf, 1, 0)

        return carry

    jax.lax.fori_loop(0, _SLOTS // _CH, chunk, 0)

    st = step_ref[...]
    use_r = st[:, _IMG:] < st[:, :_IMG]
    for c in range(_C):
        cv = canvas_ref[c]
        out_ref[0, c] = jnp.where(use_r, cv[:, _IMG:], cv[:, :_IMG])


_NV = _SORT_W // 16


def _sc_sort_body(zd_ref, pres_ref, wh_ref, sorted_ref, params_ref,
                  keys_v, pres_v, wh_v, order_v, pflat_v):
    wid = lax.axis_index("s") * 2 + lax.axis_index("c")

    @pl.when(wid < zd_ref.shape[0])
    def _():
        b = wid
        pltpu.sync_copy(zd_ref.at[b], keys_v)
        pltpu.sync_copy(pres_ref.at[b], pres_v)
        pltpu.sync_copy(wh_ref.at[b], wh_v)
        iota = lax.broadcasted_iota(jnp.int32, (16,), 0)
        keff, p2l, lidsl = [], [], []
        for v in range(_NV):
            lids = iota + 16 * v
            pa = plsc.load_gather(pres_v, [2 * lids])
            pb = plsc.load_gather(pres_v, [2 * lids + 1])
            p2 = (pa == 1) | (pb == 1)
            ke = jnp.where(p2, keys_v[pl.ds(16 * v, 16)], -jnp.inf)
            keys_v[pl.ds(16 * v, 16)] = ke
            order_v[pl.ds(16 * v, 16)] = jnp.zeros((16,), jnp.int32)
            keff.append(ke)
            p2l.append(p2)
            lidsl.append(lids)

        def mbody(m, ranks):
            km = plsc.load_gather(keys_v, [jnp.full((16,), 0, jnp.int32) + m])
            return tuple(
                ranks[v]
                + ((km > keff[v])
                   | ((km == keff[v]) & (m < lidsl[v]))).astype(jnp.int32)
                for v in range(_NV))

        ranks = lax.fori_loop(
            0, _NL, mbody,
            tuple(jnp.zeros((16,), jnp.int32) for _ in range(_NV)))

        cnt = jnp.int32(0)
        for v in range(_NV):
            mask = p2l[v] & (lidsl[v] < _NL)
            plsc.store_scatter(order_v, [ranks[v]], lidsl[v], mask=mask)
            cnt = cnt + jnp.sum(mask.astype(jnp.int32))
        plsc.store_scatter(order_v, [iota * 0 + _SLOTS],
                           jnp.full((16,), 0, jnp.int32) + cnt,
                           mask=iota == 0)

        def zbody(i, c):
            pflat_v[pl.ds(16 * i, 16)] = jnp.zeros((16,), jnp.float32)
            return c

        lax.fori_loop(0, (_SORT_W * 12) // 16, zbody, 0)

        for v in range(_NV):
            slot = lidsl[v]
            ordv = order_v[pl.ds(16 * v, 16)]
            svalid = slot < cnt
            for side in range(2):
                pj = plsc.load_gather(pres_v, [2 * ordv + side])
                ok = svalid & (pj == 1)
                off = jnp.where(ok, 0.0, 1e9)
                base = 8 * ordv + 4 * side
                cx = plsc.load_gather(wh_v, [base])
                cy = plsc.load_gather(wh_v, [base + 1])
                rw = 1.0 / (plsc.load_gather(wh_v, [base + 2]) + 1e-5)
                rh = 1.0 / (plsc.load_gather(wh_v, [base + 3]) + 1e-5)
                c0 = 5 * side
                for ci, val in ((c0, cx), (c0 + 1, cy), (c0 + 2, rw),
                                (c0 + 3, rh), (c0 + 4, off)):
                    plsc.store_scatter(pflat_v, [12 * slot + ci], val)

        pltpu.sync_copy(order_v, sorted_ref.at[b])
        pltpu.sync_copy(pflat_v, params_ref.at[b])


def _sc_sort(z_present, z_depth, z_where):
    B = z_depth.shape[0]
    zd_p = jnp.zeros((B, _SORT_W), jnp.float32).at[:, :_NL].set(z_depth[:, :, 0])
    pres_p = jnp.zeros((B, 176), jnp.int32).at[:, :170].set(z_present[:, :, 0])
    wh_p = jnp.zeros((B, 704), jnp.float32).at[:, :680].set(
        z_where.reshape(B, 680))

    run = functools.partial(
        pl.kernel,
        mesh=plsc.VectorSubcoreMesh(core_axis_name="c", subcore_axis_name="s"),
        compiler_params=pltpu.CompilerParams(needs_layout_passes=False),
        out_type=[
            jax.ShapeDtypeStruct((B, _SORT_W), jnp.int32),
            jax.ShapeDtypeStruct((B, _SORT_W * 12), jnp.float32),
        ],
        scratch_types=[
            pltpu.VMEM((_SORT_W,), jnp.float32),
            pltpu.VMEM((176,), jnp.int32),
            pltpu.VMEM((704,), jnp.float32),
            pltpu.VMEM((_SORT_W,), jnp.int32),
            pltpu.VMEM((_SORT_W * 12,), jnp.float32),
        ],
    )(_sc_sort_body)
    sorted_full, params_flat = run(zd_p, pres_p, wh_p)
    return sorted_full, params_flat.reshape(B, _SORT_W, 12)


def kernel(z_what, z_where, z_present, z_depth, indices, W_dec, b_dec):
    B, NL, Z = z_what.shape

    decoded = pl.pallas_call(
        _decode_body,
        out_shape=jax.ShapeDtypeStruct((B * NL, _C * _D * _D), jnp.float32),
        interpret=_INTERPRET,
    )(z_what.reshape(B * NL, Z), W_dec, b_dec)
    glyphs = decoded.reshape(B, NL, _C * _D, _D)

    sorted_locs, params = _sc_sort(z_present, z_depth, z_where)

    out = pl.pallas_call(
        _render_body,
        grid=(B,),
        in_specs=[
            pl.BlockSpec((1, NL, _C * _D, _D), lambda b: (b, 0, 0, 0)),
            pl.BlockSpec(memory_space=pltpu.SMEM),
            pl.BlockSpec((1, _SORT_W, 12), lambda b: (b, 0, 0)),
        ],
        out_specs=pl.BlockSpec((1, _C, _IMG, _IMG), lambda b: (b, 0, 0, 0)),
        out_shape=jax.ShapeDtypeStruct((B, _C, _IMG, _IMG), jnp.float32),
        scratch_shapes=[
            pltpu.VMEM((_C, _IMG, 2 * _IMG), jnp.float32),
            pltpu.VMEM((_IMG, 2 * _IMG), jnp.float32),
            pltpu.SMEM((1,), jnp.int32),
            pltpu.VMEM((_SORT_W, 2 * _IMG), jnp.float32),
            pltpu.VMEM((_SORT_W, _IMG), jnp.float32),
            pltpu.VMEM((_SORT_W, _IMG), jnp.float32),
        ],
        interpret=_INTERPRET,
    )(glyphs, sorted_locs, params)
    return out


# speedup vs baseline: 1.0530x; 1.0530x over previous
import functools

import jax
import jax.numpy as jnp
from jax import lax
from jax.experimental import pallas as pl
from jax.experimental.pallas import tpu as pltpu
from jax.experimental.pallas import tpu_sc as plsc

_D = 32
_IMG = 64
_C = 3
_NL = 85
_NF = 170
_SLOTS = 88
_SORT_W = 96
_CH = 4
_NV = _SORT_W // 16
_PREC = jax.lax.Precision.DEFAULT


def _sc_sort_body(zd_ref, pres_ref, wh_ref, sorted_ref, params_ref,
                  keys_v, pres_v, wh_v, order_v, pp_v):
    wid = lax.axis_index("s") * 2 + lax.axis_index("c")
    nb = sorted_ref.shape[0]

    @pl.when(wid < nb)
    def _():
        b = wid
        pltpu.sync_copy(zd_ref.at[pl.ds(b * 128, 128)], keys_v)
        pltpu.sync_copy(pres_ref, pres_v)
        pltpu.sync_copy(wh_ref.at[pl.ds(b * 8 * _NL, 8 * _NL)], wh_v)
        pbase = b * _NF
        iota = lax.broadcasted_iota(jnp.int32, (16,), 0)
        keff, p2l, lidsl = [], [], []
        for v in range(_NV):
            lids = iota + 16 * v
            pa = plsc.load_gather(
                pres_v, [pbase + jnp.minimum(2 * lids, _NF - 1)])
            pb = plsc.load_gather(
                pres_v, [pbase + jnp.minimum(2 * lids + 1, _NF - 1)])
            p2 = ((pa == 1) | (pb == 1)) & (lids < _NL)
            ke = jnp.where(p2, keys_v[pl.ds(16 * v, 16)], -jnp.inf)
            keys_v[pl.ds(16 * v, 16)] = ke
            order_v[pl.ds(16 * v, 16)] = jnp.zeros((16,), jnp.int32)
            keff.append(ke)
            p2l.append(p2)
            lidsl.append(lids)

        def mbody(m, ranks):
            km = plsc.load_gather(keys_v, [jnp.full((16,), 0, jnp.int32) + m])
            return tuple(
                ranks[v]
                + ((km > keff[v])
                   | ((km == keff[v]) & (m < lidsl[v]))).astype(jnp.int32)
                for v in range(_NV))

        ranks = lax.fori_loop(
            0, _NL, mbody,
            tuple(jnp.zeros((16,), jnp.int32) for _ in range(_NV)))

        cnt = jnp.int32(0)
        for v in range(_NV):
            plsc.store_scatter(order_v, [ranks[v]], lidsl[v], mask=p2l[v])
            cnt = cnt + jnp.sum(p2l[v].astype(jnp.int32))
        plsc.store_scatter(order_v, [iota * 0 + _SLOTS],
                           jnp.full((16,), 0, jnp.int32) + cnt,
                           mask=iota == 0)

        zero = jnp.zeros((16,), jnp.float32)
        for v in range(_NV):
            slot = lidsl[v]
            ordv = order_v[pl.ds(16 * v, 16)]
            svalid = slot < cnt
            for ci in (10, 11):
                plsc.store_scatter(pp_v, [slot, iota * 0 + ci], zero)
            for side in range(2):
                pj = plsc.load_gather(
                    pres_v, [pbase + jnp.minimum(2 * ordv + side, _NF - 1)])
                ok = svalid & (pj == 1)
                off = jnp.where(ok, 0.0, 1e9)
                base = 8 * ordv + 4 * side
                cx = plsc.load_gather(wh_v, [base])
                cy = plsc.load_gather(wh_v, [base + 1])
                rw = 1.0 / (plsc.load_gather(wh_v, [base + 2]) + 1e-5)
                rh = 1.0 / (plsc.load_gather(wh_v, [base + 3]) + 1e-5)
                c0 = 5 * side
                for ci, val in ((c0, cx), (c0 + 1, cy), (c0 + 2, rw),
                                (c0 + 3, rh), (c0 + 4, off)):
                    plsc.store_scatter(pp_v, [slot, iota * 0 + ci], val)

        pltpu.sync_copy(order_v, sorted_ref.at[b])
        pltpu.sync_copy(pp_v, params_ref.at[b])


def _sc_sort(z_present, z_depth, z_where):
    B = z_depth.shape[0]
    zd_p = jnp.zeros((B, 128), jnp.float32).at[:, :_NL].set(
        z_depth[:, :, 0]).reshape(B * 128)
    pres_f = z_present.reshape(B * _NF)
    wh_f = z_where.reshape(B * 8 * _NL)

    run = functools.partial(
        pl.kernel,
        mesh=plsc.VectorSubcoreMesh(core_axis_name="c", subcore_axis_name="s"),
        compiler_params=pltpu.CompilerParams(needs_layout_passes=False),
        out_type=[
            jax.ShapeDtypeStruct((B, _SORT_W), jnp.int32),
            jax.ShapeDtypeStruct((B, _SORT_W, 12), jnp.float32),
        ],
        scratch_types=[
            pltpu.VMEM((128,), jnp.float32),
            pltpu.VMEM((4 * _NF,), jnp.int32),
            pltpu.VMEM((8 * _NL,), jnp.float32),
            pltpu.VMEM((_SORT_W,), jnp.int32),
            pltpu.VMEM((_SORT_W, 12), jnp.float32),
        ],
    )(_sc_sort_body)
    return run(zd_p, pres_f, wh_f)


def _decode_body(zw_ref, w_ref, b_ref, out_ref):
    x = jnp.dot(zw_ref[...], w_ref[...],
                preferred_element_type=jnp.float32, precision=_PREC)
    out_ref[...] = jax.nn.sigmoid(x + b_ref[...][None, :])


def _render_body(glyphs_ref, sorted_ref, params_ref,
                 out_ref, canvas_ref, step_ref, done_ref,
                 sxall_ref, syar_ref, sybr_ref):
    b = pl.program_id(0)
    canvas_ref[...] = jnp.zeros(canvas_ref.shape, jnp.float32)
    step_ref[...] = jnp.full(step_ref.shape, jnp.inf, jnp.float32)
    done_ref[0] = 0
    cnt = sorted_ref[b, _SLOTS]

    lane = jax.lax.broadcasted_iota(jnp.int32, (1, 2 * _IMG), 1)
    gxpair = (lane % _IMG).astype(jnp.float32) * (2.0 / (_IMG - 1)) - 1.0
    selx = lane < _IMG
    gyrow = jax.lax.broadcasted_iota(jnp.int32, (1, _IMG), 1).astype(jnp.float32) * (2.0 / (_IMG - 1)) - 1.0
    xp = jax.lax.broadcasted_iota(jnp.int32, (_D, 1), 0).astype(jnp.float32)
    kcol = jax.lax.broadcasted_iota(jnp.int32, (_IMG, 1), 0)
    ksel = kcol < _D
    kmodcol = (kcol % _D).astype(jnp.float32)
    lmaskA = selx.astype(jnp.float32) * jnp.ones((_D, 1), jnp.float32)
    lmaskB = 1.0 - lmaskA
    leftmask = jnp.broadcast_to(selx, (_IMG, 2 * _IMG))
    half = (_D - 1) / 2.0

    pr = params_ref[0]
    cxa, cya, rwa, rha, offa = (pr[:, i:i + 1] for i in range(5))
    cxb, cyb, rwb, rhb, offb = (pr[:, i:i + 1] for i in range(5, 10))
    sx_a = ((gxpair - (2.0 * cxa - 1.0)) * rwa + 1.0) * half + offa
    sx_b = ((gxpair - (2.0 * cxb - 1.0)) * rwb + 1.0) * half + offb
    sxall_ref[...] = jnp.where(selx, sx_a, sx_b)
    syar_ref[...] = ((gyrow - (2.0 * cya - 1.0)) * rha + 1.0) * half + offa
    sybr_ref[...] = ((gyrow - (2.0 * cyb - 1.0)) * rhb + 1.0) * half + offb

    def chunk(ci, carry):
        @pl.when((ci * _CH < cnt) & (done_ref[0] == 0))
        def _():
            st = step_ref[...]
            cv = [canvas_ref[c] for c in range(_C)]
            new_st = st
            for u in range(_CH):
                k = ci * _CH + u
                loc = sorted_ref[b, k]
                sx_row = sxall_ref[pl.ds(k, 1), :]
                rxt2 = jnp.maximum(0.0, 1.0 - jnp.abs(sx_row - xp))
                sya = syar_ref[pl.ds(k, 1), :]
                syb = sybr_ref[pl.ds(k, 1), :]
                sy_sel = jnp.where(ksel, sya, syb)
                rycat_t = jnp.maximum(0.0, 1.0 - jnp.abs(sy_sel - kmodcol))

                g = glyphs_ref[0, loc]
                a2 = jnp.dot(g, rxt2, preferred_element_type=jnp.float32,
                             precision=_PREC)
                r2 = []
                for c in range(_C):
                    a2c = a2[c * _D:(c + 1) * _D, :]
                    aext = jnp.concatenate([a2c * lmaskA, a2c * lmaskB], axis=0)
                    r2.append(jax.lax.dot_general(
                        rycat_t, aext, (((0,), (0,)), ((), ())),
                        preferred_element_type=jnp.float32,
                        precision=_PREC))
                upd = (r2[0] != 0.0) & (new_st == jnp.inf)
                for c in range(_C):
                    cv[c] = jnp.where(upd, r2[c], cv[c])
                new_st = jnp.where(upd, jnp.float32(k), new_st)
            for c in range(_C):
                canvas_ref[c] = cv[c]
            step_ref[...] = new_st
            maxleft = jnp.max(jnp.where(leftmask, new_st, -jnp.inf))
            done_ref[0] = jnp.where(maxleft < jnp.inf, 1, 0)

        return carry

    jax.lax.fori_loop(0, _SLOTS // _CH, chunk, 0)

    st = step_ref[...]
    use_r = st[:, _IMG:] < st[:, :_IMG]
    for c in range(_C):
        cv = canvas_ref[c]
        out_ref[0, c] = jnp.where(use_r, cv[:, _IMG:], cv[:, :_IMG])


def kernel(z_what, z_where, z_present, z_depth, indices, W_dec, b_dec):
    B, NL, Z = z_what.shape

    sorted_locs, params = _sc_sort(z_present, z_depth, z_where)

    decoded = pl.pallas_call(
        _decode_body,
        out_shape=jax.ShapeDtypeStruct((B * NL, _C * _D * _D), jnp.float32),
    )(z_what.reshape(B * NL, Z), W_dec, b_dec)
    glyphs = decoded.reshape(B, NL, _C * _D, _D)

    out = pl.pallas_call(
        _render_body,
        grid=(B,),
        in_specs=[
            pl.BlockSpec((1, NL, _C * _D, _D), lambda b: (b, 0, 0, 0)),
            pl.BlockSpec(memory_space=pltpu.SMEM),
            pl.BlockSpec((1, _SORT_W, 12), lambda b: (b, 0, 0)),
        ],
        out_specs=pl.BlockSpec((1, _C, _IMG, _IMG), lambda b: (b, 0, 0, 0)),
        out_shape=jax.ShapeDtypeStruct((B, _C, _IMG, _IMG), jnp.float32),
        scratch_shapes=[
            pltpu.VMEM((_C, _IMG, 2 * _IMG), jnp.float32),
            pltpu.VMEM((_IMG, 2 * _IMG), jnp.float32),
            pltpu.SMEM((1,), jnp.int32),
            pltpu.VMEM((_SORT_W, 2 * _IMG), jnp.float32),
            pltpu.VMEM((_SORT_W, _IMG), jnp.float32),
            pltpu.VMEM((_SORT_W, _IMG), jnp.float32),
        ],
    )(glyphs, sorted_locs, params)
    return out


# speedup vs baseline: 1.0976x; 1.0423x over previous
import functools

import jax
import jax.numpy as jnp
from jax import lax
from jax.experimental import pallas as pl
from jax.experimental.pallas import tpu as pltpu
from jax.experimental.pallas import tpu_sc as plsc

_D = 32
_IMG = 64
_C = 3
_NL = 85
_NF = 170
_SLOTS = 88
_SORT_W = 96
_CH = 8
_NV = _SORT_W // 16
_PREC = jax.lax.Precision.DEFAULT


def _sc_sort_body(zd_ref, pres_ref, wh_ref, sorted_ref, params_ref,
                  keys_v, pres_v, wh_v, order_v, pp_v):
    wid = lax.axis_index("s") * 2 + lax.axis_index("c")
    nb = sorted_ref.shape[0]

    @pl.when(wid < nb)
    def _():
        b = wid
        pltpu.sync_copy(zd_ref.at[pl.ds(b * 128, 128)], keys_v)
        pltpu.sync_copy(pres_ref, pres_v)
        pltpu.sync_copy(wh_ref.at[pl.ds(b * 8 * _NL, 8 * _NL)], wh_v)
        pbase = b * _NF
        iota = lax.broadcasted_iota(jnp.int32, (16,), 0)
        keff, p2l, lidsl = [], [], []
        for v in range(_NV):
            lids = iota + 16 * v
            pa = plsc.load_gather(
                pres_v, [pbase + jnp.minimum(2 * lids, _NF - 1)])
            pb = plsc.load_gather(
                pres_v, [pbase + jnp.minimum(2 * lids + 1, _NF - 1)])
            p2 = ((pa == 1) | (pb == 1)) & (lids < _NL)
            ke = jnp.where(p2, keys_v[pl.ds(16 * v, 16)], -jnp.inf)
            keys_v[pl.ds(16 * v, 16)] = ke
            order_v[pl.ds(16 * v, 16)] = jnp.zeros((16,), jnp.int32)
            keff.append(ke)
            p2l.append(p2)
            lidsl.append(lids)

        def mbody(m, ranks):
            km = plsc.load_gather(keys_v, [jnp.full((16,), 0, jnp.int32) + m])
            return tuple(
                ranks[v]
                + ((km > keff[v])
                   | ((km == keff[v]) & (m < lidsl[v]))).astype(jnp.int32)
                for v in range(_NV))

        ranks = lax.fori_loop(
            0, _NL, mbody,
            tuple(jnp.zeros((16,), jnp.int32) for _ in range(_NV)))

        cnt = jnp.int32(0)
        for v in range(_NV):
            plsc.store_scatter(order_v, [ranks[v]], lidsl[v], mask=p2l[v])
            cnt = cnt + jnp.sum(p2l[v].astype(jnp.int32))
        plsc.store_scatter(order_v, [iota * 0 + _SLOTS],
                           jnp.full((16,), 0, jnp.int32) + cnt,
                           mask=iota == 0)

        zero = jnp.zeros((16,), jnp.float32)
        for v in range(_NV):
            slot = lidsl[v]
            ordv = order_v[pl.ds(16 * v, 16)]
            svalid = slot < cnt
            for ci in (10, 11):
                plsc.store_scatter(pp_v, [slot, iota * 0 + ci], zero)
            for side in range(2):
                pj = plsc.load_gather(
                    pres_v, [pbase + jnp.minimum(2 * ordv + side, _NF - 1)])
                ok = svalid & (pj == 1)
                off = jnp.where(ok, 0.0, 1e9)
                base = 8 * ordv + 4 * side
                cx = plsc.load_gather(wh_v, [base])
                cy = plsc.load_gather(wh_v, [base + 1])
                rw = 1.0 / (plsc.load_gather(wh_v, [base + 2]) + 1e-5)
                rh = 1.0 / (plsc.load_gather(wh_v, [base + 3]) + 1e-5)
                c0 = 5 * side
                for ci, val in ((c0, cx), (c0 + 1, cy), (c0 + 2, rw),
                                (c0 + 3, rh), (c0 + 4, off)):
                    plsc.store_scatter(pp_v, [slot, iota * 0 + ci], val)

        pltpu.sync_copy(order_v, sorted_ref.at[b])
        pltpu.sync_copy(pp_v, params_ref.at[b])


def _sc_sort(z_present, z_depth, z_where):
    B = z_depth.shape[0]
    zd_p = jnp.zeros((B, 128), jnp.float32).at[:, :_NL].set(
        z_depth[:, :, 0]).reshape(B * 128)
    pres_f = z_present.reshape(B * _NF)
    wh_f = z_where.reshape(B * 8 * _NL)

    run = functools.partial(
        pl.kernel,
        mesh=plsc.VectorSubcoreMesh(core_axis_name="c", subcore_axis_name="s"),
        compiler_params=pltpu.CompilerParams(needs_layout_passes=False),
        out_type=[
            jax.ShapeDtypeStruct((B, _SORT_W), jnp.int32),
            jax.ShapeDtypeStruct((B, _SORT_W, 12), jnp.float32),
        ],
        scratch_types=[
            pltpu.VMEM((128,), jnp.float32),
            pltpu.VMEM((4 * _NF,), jnp.int32),
            pltpu.VMEM((8 * _NL,), jnp.float32),
            pltpu.VMEM((_SORT_W,), jnp.int32),
            pltpu.VMEM((_SORT_W, 12), jnp.float32),
        ],
    )(_sc_sort_body)
    return run(zd_p, pres_f, wh_f)


def _decode_body(zw_ref, w_ref, b_ref, out_ref):
    x = jnp.dot(zw_ref[...], w_ref[...],
                preferred_element_type=jnp.float32, precision=_PREC)
    out_ref[...] = jax.nn.sigmoid(x + b_ref[...][None, :])


def _render_body(glyphs_ref, sorted_ref, params_ref,
                 out_ref, canvas_ref, step_ref, done_ref,
                 sxall_ref, syar_ref, sybr_ref):
    b = pl.program_id(0)
    canvas_ref[...] = jnp.zeros(canvas_ref.shape, jnp.float32)
    step_ref[...] = jnp.full(step_ref.shape, jnp.inf, jnp.float32)
    done_ref[0] = 0
    cnt = sorted_ref[b, _SLOTS]

    lane = jax.lax.broadcasted_iota(jnp.int32, (1, 2 * _IMG), 1)
    gxpair = (lane % _IMG).astype(jnp.float32) * (2.0 / (_IMG - 1)) - 1.0
    selx = lane < _IMG
    gyrow = jax.lax.broadcasted_iota(jnp.int32, (1, _IMG), 1).astype(jnp.float32) * (2.0 / (_IMG - 1)) - 1.0
    xp = jax.lax.broadcasted_iota(jnp.int32, (_D, 1), 0).astype(jnp.float32)
    kcol = jax.lax.broadcasted_iota(jnp.int32, (_IMG, 1), 0)
    ksel = kcol < _D
    kmodcol = (kcol % _D).astype(jnp.float32)
    lmaskA = selx.astype(jnp.float32) * jnp.ones((_D, 1), jnp.float32)
    lmaskB = 1.0 - lmaskA
    leftmask = jnp.broadcast_to(selx, (_IMG, 2 * _IMG))
    half = (_D - 1) / 2.0

    pr = params_ref[0]
    cxa, cya, rwa, rha, offa = (pr[:, i:i + 1] for i in range(5))
    cxb, cyb, rwb, rhb, offb = (pr[:, i:i + 1] for i in range(5, 10))
    sx_a = ((gxpair - (2.0 * cxa - 1.0)) * rwa + 1.0) * half + offa
    sx_b = ((gxpair - (2.0 * cxb - 1.0)) * rwb + 1.0) * half + offb
    sxall_ref[...] = jnp.where(selx, sx_a, sx_b)
    syar_ref[...] = ((gyrow - (2.0 * cya - 1.0)) * rha + 1.0) * half + offa
    sybr_ref[...] = ((gyrow - (2.0 * cyb - 1.0)) * rhb + 1.0) * half + offb

    def chunk(ci, carry):
        @pl.when((ci * _CH < cnt) & (done_ref[0] == 0))
        def _():
            st = step_ref[...]
            cv = [canvas_ref[c] for c in range(_C)]
            new_st = st
            for u in range(_CH):
                k = ci * _CH + u
                loc = sorted_ref[b, k]
                sx_row = sxall_ref[pl.ds(k, 1), :]
                rxt2 = jnp.maximum(0.0, 1.0 - jnp.abs(sx_row - xp))
                sya = syar_ref[pl.ds(k, 1), :]
                syb = sybr_ref[pl.ds(k, 1), :]
                sy_sel = jnp.where(ksel, sya, syb)
                rycat_t = jnp.maximum(0.0, 1.0 - jnp.abs(sy_sel - kmodcol))

                g = glyphs_ref[0, loc]
                a2 = jnp.dot(g, rxt2, preferred_element_type=jnp.float32,
                             precision=_PREC)
                r2 = []
                for c in range(_C):
                    a2c = a2[c * _D:(c + 1) * _D, :]
                    aext = jnp.concatenate([a2c * lmaskA, a2c * lmaskB], axis=0)
                    r2.append(jax.lax.dot_general(
                        rycat_t, aext, (((0,), (0,)), ((), ())),
                        preferred_element_type=jnp.float32,
                        precision=_PREC))
                upd = (r2[0] != 0.0) & (new_st == jnp.inf)
                for c in range(_C):
                    cv[c] = jnp.where(upd, r2[c], cv[c])
                new_st = jnp.where(upd, jnp.float32(k), new_st)
            for c in range(_C):
                canvas_ref[c] = cv[c]
            step_ref[...] = new_st
            maxleft = jnp.max(jnp.where(leftmask, new_st, -jnp.inf))
            done_ref[0] = jnp.where(maxleft < jnp.inf, 1, 0)

        return carry

    jax.lax.fori_loop(0, _SLOTS // _CH, chunk, 0)

    st = step_ref[...]
    use_r = st[:, _IMG:] < st[:, :_IMG]
    for c in range(_C):
        cv = canvas_ref[c]
        out_ref[0, c] = jnp.where(use_r, cv[:, _IMG:], cv[:, :_IMG])


def kernel(z_what, z_where, z_present, z_depth, indices, W_dec, b_dec):
    B, NL, Z = z_what.shape

    sorted_locs, params = _sc_sort(z_present, z_depth, z_where)

    decoded = pl.pallas_call(
        _decode_body,
        out_shape=jax.ShapeDtypeStruct((B * NL, _C * _D * _D), jnp.float32),
    )(z_what.reshape(B * NL, Z), W_dec, b_dec)
    glyphs = decoded.reshape(B, NL, _C * _D, _D)

    out = pl.pallas_call(
        _render_body,
        grid=(B,),
        in_specs=[
            pl.BlockSpec((1, NL, _C * _D, _D), lambda b: (b, 0, 0, 0)),
            pl.BlockSpec(memory_space=pltpu.SMEM),
            pl.BlockSpec((1, _SORT_W, 12), lambda b: (b, 0, 0)),
        ],
        out_specs=pl.BlockSpec((1, _C, _IMG, _IMG), lambda b: (b, 0, 0, 0)),
        out_shape=jax.ShapeDtypeStruct((B, _C, _IMG, _IMG), jnp.float32),
        scratch_shapes=[
            pltpu.VMEM((_C, _IMG, 2 * _IMG), jnp.float32),
            pltpu.VMEM((_IMG, 2 * _IMG), jnp.float32),
            pltpu.SMEM((1,), jnp.int32),
            pltpu.VMEM((_SORT_W, 2 * _IMG), jnp.float32),
            pltpu.VMEM((_SORT_W, _IMG), jnp.float32),
            pltpu.VMEM((_SORT_W, _IMG), jnp.float32),
        ],
    )(glyphs, sorted_locs, params)
    return out
